# baseline (speedup 1.0000x reference)
import functools

import jax
import jax.numpy as jnp
from jax import lax
from jax.experimental import pallas as pl
from jax.experimental.pallas import tpu as pltpu
from jax.experimental.pallas import tpu_sc as plsc

_NC = 1
_NS = 16
_NW = _NC * _NS
_L = 16

_B = 64
_N = 2048
_D0 = 256
_D1 = 64
_BPW = _B // _NW


def _sc_body(scores_hbm, attr0_hbm, attr1_hbm, out0_hbm, out1_hbm,
             scores_v, rows0_v, rows1_v, col1_v, sem0, sem1):
  wid = lax.axis_index("s") * _NC + lax.axis_index("c")
  base = wid * _BPW

  pltpu.sync_copy(scores_hbm.at[pl.ds(base * _N, _BPW * _N)], scores_v)

  lane = lax.broadcasted_iota(jnp.int32, (_L,), 0)

  copies = []
  offs = []

  def argmax_pair(bs):
    def body(i, carry):
      pos = i * _L + lane
      out = []
      for k, b in enumerate(bs):
        best, bidx = carry[2 * k], carry[2 * k + 1]
        v = scores_v[pl.ds(b * _N + i * _L, _L)]
        take = v > best
        out.append(jnp.where(take, v, best))
        out.append(jnp.where(take, pos, bidx))
      return tuple(out)

    init = (jnp.full((_L,), -jnp.inf, jnp.float32),
            jnp.zeros((_L,), jnp.int32)) * len(bs)
    return lax.fori_loop(0, _N // _L, body, init, unroll=1)

  def issue_gathers(bs, res):
    for k, b in enumerate(bs):
      best, bidx = res[2 * k], res[2 * k + 1]
      sv, _ = plsc.sort_key_val(best, bidx, descending=True)
      m = sv[0]
      cand = jnp.where(best == m, bidx, jnp.int32(_N))
      ci, _ = plsc.sort_key_val(cand, cand)
      idx = ci[0]
      bg = base + b
      off = (idx // 128) * 128
      offs.append(idx - off)
      copies.append(pltpu.async_copy(
          attr0_hbm.at[bg * _N + idx], rows0_v.at[b], sem0))
      copies.append(pltpu.async_copy(
          attr1_hbm.at[bg, :, pl.ds(off, 128)], col1_v.at[b], sem1))

  half = _BPW // 2
  res = argmax_pair(tuple(range(half)))
  issue_gathers(tuple(range(half)), res)
  res = argmax_pair(tuple(range(half, _BPW)))
  issue_gathers(tuple(range(half, _BPW)), res)
  for cp in copies:
    cp.wait()
  for b in range(_BPW):
    sel = jnp.full((_L,), offs[b], jnp.int32)
    for j in range(_D1 // _L):
      v = plsc.load_gather(col1_v, [jnp.full((_L,), b, jnp.int32),
                                    j * _L + lane, sel])
      rows1_v[b, pl.ds(j * _L, _L)] = v
  pltpu.sync_copy(rows0_v, out0_hbm.at[pl.ds(base, _BPW)])
  pltpu.sync_copy(rows1_v, out1_hbm.at[pl.ds(base, _BPW)])


@jax.jit
def kernel(scores, attr0, attr1):
  scores1 = scores.reshape(_B * _N)
  a0 = attr0.reshape(_B * _N, _D0)
  a1t = attr1.transpose(0, 2, 1)

  mesh = plsc.VectorSubcoreMesh(core_axis_name="c", subcore_axis_name="s",
                                num_cores=1)
  run = pl.kernel(
      _sc_body,
      out_type=(jax.ShapeDtypeStruct((_B, _D0), jnp.float32),
                jax.ShapeDtypeStruct((_B, _D1), jnp.float32)),
      mesh=mesh,
      scratch_types=[
          pltpu.VMEM((_BPW * _N,), jnp.float32),
          pltpu.VMEM((_BPW, _D0), jnp.float32),
          pltpu.VMEM((_BPW, _D1), jnp.float32),
          pltpu.VMEM((_BPW, _D1, 128), jnp.float32),
          pltpu.SemaphoreType.DMA,
          pltpu.SemaphoreType.DMA,
      ],
      compiler_params=pltpu.CompilerParams(needs_layout_passes=False),
  )
  return run(scores1, a0, a1t)


# speedup vs baseline: 1.0131x; 1.0131x over previous
import functools

import jax
import jax.numpy as jnp
from jax import lax
from jax.experimental import pallas as pl
from jax.experimental.pallas import tpu as pltpu
from jax.experimental.pallas import tpu_sc as plsc

_NC = 1
_NS = 16
_NW = _NC * _NS
_L = 16

_B = 64
_N = 2048
_D0 = 256
_D1 = 64
_BPW = _B // _NW


def _sc_body(scores_hbm, attr0_hbm, attr1_hbm, out0_hbm, out1_hbm,
             scores_v, rows0_v, rows1_v, col1_v, sem0, sem1):
  wid = lax.axis_index("s") * _NC + lax.axis_index("c")
  base = wid * _BPW

  pltpu.sync_copy(scores_hbm.at[pl.ds(base * _N, _BPW * _N)], scores_v)

  lane = lax.broadcasted_iota(jnp.int32, (_L,), 0)

  copies = []
  offs = []

  def argmax_pair(bs):
    def body(i, carry):
      pos = i * _L + lane
      out = []
      for k, b in enumerate(bs):
        best, bidx = carry[2 * k], carry[2 * k + 1]
        v = scores_v[pl.ds(b * _N + i * _L, _L)]
        take = v > best
        out.append(jnp.where(take, v, best))
        out.append(jnp.where(take, pos, bidx))
      return tuple(out)

    init = (jnp.full((_L,), -jnp.inf, jnp.float32),
            jnp.zeros((_L,), jnp.int32)) * len(bs)
    return lax.fori_loop(0, _N // _L, body, init, unroll=8)

  def issue_gathers(bs, res):
    for k, b in enumerate(bs):
      best, bidx = res[2 * k], res[2 * k + 1]
      sv, _ = plsc.sort_key_val(best, bidx, descending=True)
      m = sv[0]
      cand = jnp.where(best == m, bidx, jnp.int32(_N))
      ci, _ = plsc.sort_key_val(cand, cand)
      idx = ci[0]
      bg = base + b
      off = (idx // 128) * 128
      offs.append(idx - off)
      copies.append(pltpu.async_copy(
          attr0_hbm.at[bg * _N + idx], rows0_v.at[b], sem0))
      copies.append(pltpu.async_copy(
          attr1_hbm.at[bg, :, pl.ds(off, 128)], col1_v.at[b], sem1))

  half = _BPW // 2
  res = argmax_pair(tuple(range(half)))
  issue_gathers(tuple(range(half)), res)
  res = argmax_pair(tuple(range(half, _BPW)))
  issue_gathers(tuple(range(half, _BPW)), res)
  for cp in copies:
    cp.wait()
  for b in range(_BPW):
    sel = jnp.full((_L,), offs[b], jnp.int32)
    for j in range(_D1 // _L):
      v = plsc.load_gather(col1_v, [jnp.full((_L,), b, jnp.int32),
                                    j * _L + lane, sel])
      rows1_v[b, pl.ds(j * _L, _L)] = v
  c0 = pltpu.async_copy(rows0_v, out0_hbm.at[pl.ds(base, _BPW)], sem0)
  c1 = pltpu.async_copy(rows1_v, out1_hbm.at[pl.ds(base, _BPW)], sem1)
  c0.wait()
  c1.wait()


@jax.jit
def kernel(scores, attr0, attr1):
  scores1 = scores.reshape(_B * _N)
  a0 = attr0.reshape(_B * _N, _D0)
  a1t = attr1.transpose(0, 2, 1)

  mesh = plsc.VectorSubcoreMesh(core_axis_name="c", subcore_axis_name="s",
                                num_cores=1)
  run = pl.kernel(
      _sc_body,
      out_type=(jax.ShapeDtypeStruct((_B, _D0), jnp.float32),
                jax.ShapeDtypeStruct((_B, _D1), jnp.float32)),
      mesh=mesh,
      scratch_types=[
          pltpu.VMEM((_BPW * _N,), jnp.float32),
          pltpu.VMEM((_BPW, _D0), jnp.float32),
          pltpu.VMEM((_BPW, _D1), jnp.float32),
          pltpu.VMEM((_BPW, _D1, 128), jnp.float32),
          pltpu.SemaphoreType.DMA,
          pltpu.SemaphoreType.DMA,
      ],
      compiler_params=pltpu.CompilerParams(needs_layout_passes=False),
  )
  return run(scores1, a0, a1t)
